# baseline (speedup 1.0000x reference)
import functools

import jax
import jax.numpy as jnp
from jax import lax
from jax.experimental import pallas as pl
from jax.experimental.pallas import tpu as pltpu
from jax.experimental.pallas import tpu_sc as plsc

N_NODES = 10000
ADJ_W = 32
EMB = 128
HID = 128
LAYERS = 3
SENT = 100
TLEN = 100

NC = 2
NS = 16
NW = NC * NS


def _sc_gather_small(V, D, n_chunks, chunk, dtype):
    rows_pw = n_chunks * chunk
    mesh = plsc.VectorSubcoreMesh(core_axis_name="c", subcore_axis_name="s")

    @functools.partial(
        pl.kernel,
        out_type=jax.ShapeDtypeStruct((NW * rows_pw, D), dtype),
        mesh=mesh,
        scratch_types=[
            pltpu.VMEM((n_chunks, chunk), jnp.int32),
            pltpu.VMEM((rows_pw, D), dtype),
            pltpu.SemaphoreType.DMA,
        ],
    )
    def k(table_h, idx_h, out_h, idx_v, rows_v, sem):
        wid = lax.axis_index("s") * NC + lax.axis_index("c")
        base = wid * rows_pw
        pltpu.sync_copy(idx_h.at[wid], idx_v)
        handles = []
        for ci in range(n_chunks):
            handles.append(
                pltpu.async_copy(
                    table_h.at[idx_v.at[ci]],
                    rows_v.at[pl.ds(ci * chunk, chunk)],
                    sem,
                )
            )
        for h in handles:
            h.wait()
        pltpu.sync_copy(rows_v, out_h.at[pl.ds(base, rows_pw)])

    return k


def _sc_gather_big(V, D, n_chunks, chunk, dtype):
    rows_pw = n_chunks * chunk
    mesh = plsc.VectorSubcoreMesh(core_axis_name="c", subcore_axis_name="s")
    NB = 3
    assert n_chunks % NB == 0
    groups = n_chunks // NB
    assert groups >= 2

    assert groups % 2 == 0

    @functools.partial(
        pl.kernel,
        out_type=jax.ShapeDtypeStruct((NW * rows_pw, D), dtype),
        mesh=mesh,
        scratch_types=[
            pltpu.VMEM((n_chunks, chunk), jnp.int32),
            pltpu.VMEM((2 * NB, chunk, D), dtype),
            pltpu.SemaphoreType.DMA,
            pltpu.SemaphoreType.DMA,
            pltpu.SemaphoreType.DMA,
        ],
    )
    def k(table_h, idx_h, out_h, idx_v, rows_v, gsem, osem_a, osem_b):
        wid = lax.axis_index("s") * NC + lax.axis_index("c")
        base = wid * rows_pw
        pltpu.sync_copy(idx_h.at[wid], idx_v)

        def gfire(ci, buf):
            pltpu.async_copy(table_h.at[idx_v.at[ci]], rows_v.at[buf], gsem)

        def gdrain(buf):
            pltpu.make_async_copy(
                table_h.at[idx_v.at[0]], rows_v.at[buf], gsem).wait()

        def ofire(ci, buf, osem):
            pltpu.async_copy(
                rows_v.at[buf], out_h.at[pl.ds(base + ci * chunk, chunk)], osem)

        def odrain(buf, osem):
            pltpu.make_async_copy(
                rows_v.at[buf], out_h.at[pl.ds(base, chunk)], osem).wait()

        for b in range(NB):
            gfire(b, b)

        def seg(g, s, osem_this, osem_other):
            c0 = g * NB
            o = NB - s
            for b in range(NB):
                gdrain(s + b)
            for b in range(NB):
                ofire(c0 + b, s + b, osem_this)
            @pl.when(g > 0)
            def _():
                for b in range(NB):
                    odrain(o + b, osem_other)
            @pl.when(g < groups - 1)
            def _():
                for b in range(NB):
                    gfire(c0 + NB + b, o + b)

        def body(g, carry):
            even = lax.rem(g, 2) == 0
            @pl.when(even)
            def _():
                seg(g, 0, osem_a, osem_b)
            @pl.when(jnp.logical_not(even))
            def _():
                seg(g, NB, osem_b, osem_a)
            return carry

        lax.fori_loop(0, groups, body, 0)
        for b in range(NB):
            odrain(NB + b, osem_b)

    return k


def _sc_gather_sum(V, n_chunks, chunk):
    D = HID
    rows_pw = n_chunks * chunk
    rows_out = rows_pw // ADJ_W
    mesh = plsc.VectorSubcoreMesh(core_axis_name="c", subcore_axis_name="s")
    NB = 2
    assert n_chunks % (2 * NB) == 0
    groups = n_chunks // NB

    @functools.partial(
        pl.kernel,
        out_type=jax.ShapeDtypeStruct((NW * rows_out, D), jnp.float32),
        mesh=mesh,
        scratch_types=[
            pltpu.VMEM((n_chunks, chunk), jnp.int32),
            pltpu.VMEM((n_chunks, chunk), jnp.int32),
            pltpu.VMEM((2 * NB, chunk, D), jnp.float32),
            pltpu.VMEM_SHARED((NS * rows_out, D), jnp.float32),
            pltpu.SemaphoreType.DMA,
            pltpu.SemaphoreType.DMA,
            pltpu.SemaphoreType.DMA,
        ],
    )
    def k(table_h, idx_h, scat_h, zer_h, out_h, idx_v, scat_v, rows_v, acc_s,
          gsem, osem_a, osem_b):
        cid = lax.axis_index("c")
        sid = lax.axis_index("s")
        wid = sid * NC + cid
        pltpu.sync_copy(idx_h.at[wid], idx_v)
        pltpu.sync_copy(scat_h.at[wid], scat_v)
        pltpu.sync_copy(zer_h, acc_s.at[pl.ds(sid * rows_out, rows_out)])

        def gfire(ci, buf):
            pltpu.async_copy(table_h.at[idx_v.at[ci]], rows_v.at[buf], gsem)

        def gdrain(buf):
            pltpu.make_async_copy(
                table_h.at[idx_v.at[0]], rows_v.at[buf], gsem).wait()

        def ofire(ci, buf, osem):
            pltpu.async_copy(
                rows_v.at[buf], acc_s.at[scat_v.at[ci]], osem, add=True)

        def odrain(buf, osem):
            pltpu.make_async_copy(
                rows_v.at[buf], acc_s.at[scat_v.at[0]], osem).wait()

        for b in range(NB):
            gfire(b, b)

        def seg(g, s, osem_this, osem_other):
            c0 = g * NB
            o = NB - s
            for b in range(NB):
                gdrain(s + b)
            for b in range(NB):
                ofire(c0 + b, s + b, osem_this)
            @pl.when(g > 0)
            def _():
                for b in range(NB):
                    odrain(o + b, osem_other)
            @pl.when(g < groups - 1)
            def _():
                for b in range(NB):
                    gfire(c0 + NB + b, o + b)

        def body(g, carry):
            even = lax.rem(g, 2) == 0
            @pl.when(even)
            def _():
                seg(g, 0, osem_a, osem_b)
            @pl.when(jnp.logical_not(even))
            def _():
                seg(g, NB, osem_b, osem_a)
            return carry

        lax.fori_loop(0, groups, body, 0)
        for b in range(NB):
            odrain(NB + b, osem_b)
        pltpu.sync_copy(acc_s.at[pl.ds(sid * rows_out, rows_out)],
                        out_h.at[pl.ds(wid * rows_out, rows_out)])

    return k


def _gather_sum_rows(table, idx_flat):
    B = idx_flat.shape[0]
    V, D = table.shape
    chunk = 128
    n_chunks = -(-B // (NW * chunk))
    n_chunks = ((n_chunks + 3) // 4) * 4
    rows_pw = n_chunks * chunk
    rows_out = rows_pw // ADJ_W
    fn = _sc_gather_sum(V, n_chunks, chunk)
    idx3 = _pad_idx(idx_flat, n_chunks, chunk)
    base = (jnp.arange(rows_pw, dtype=jnp.int32) // ADJ_W).reshape(
        n_chunks, chunk)
    scat3 = base[None] + ((jnp.arange(NW, dtype=jnp.int32) // NC)
                          * rows_out)[:, None, None]
    zer = jnp.zeros((rows_out, D), jnp.float32)
    out = fn(table, idx3, scat3, zer)
    return out[:B // ADJ_W]


def _pad_idx(idx_flat, n_chunks, chunk):
    total = NW * n_chunks * chunk
    idx_flat = idx_flat.astype(jnp.int32)
    pad = total - idx_flat.shape[0]
    if pad:
        idx_flat = jnp.concatenate([idx_flat, jnp.zeros((pad,), jnp.int32)])
    return idx_flat.reshape(NW, n_chunks, chunk)


def _gather_rows(table, idx_flat, big=False):
    B = idx_flat.shape[0]
    V, D = table.shape
    chunk = 128 if big else 64
    n_chunks = -(-B // (NW * chunk))
    if big:
        n_chunks = ((n_chunks + 5) // 6) * 6
        fn = _sc_gather_big(V, D, n_chunks, chunk, table.dtype)
    else:
        fn = _sc_gather_small(V, D, n_chunks, chunk, table.dtype)
    idx3 = _pad_idx(idx_flat, n_chunks, chunk)
    out = fn(table, idx3)
    return out[:B]


def _lstm_body(emb_ref, wif_ref, whf_ref, bf_ref, wib_ref, whb_ref, bb_ref,
               outf_ref, outb_ref):
    H2 = HID // 2

    def run(wi_ref, wh_ref, b_ref, out_ref, reverse):
        def step(s, carry):
            h, c = carry
            t = (TLEN - 1 - s) if reverse else s
            xt = emb_ref[t]
            g = (jnp.dot(xt, wi_ref[...], preferred_element_type=jnp.float32)
                 + jnp.dot(h, wh_ref[...], preferred_element_type=jnp.float32)
                 + b_ref[...])
            i = jax.nn.sigmoid(g[:, 0:H2])
            f = jax.nn.sigmoid(g[:, H2:2 * H2])
            gg = jnp.tanh(g[:, 2 * H2:3 * H2])
            o = jax.nn.sigmoid(g[:, 3 * H2:4 * H2])
            c = f * c + i * gg
            h = o * jnp.tanh(c)
            out_ref[t] = h
            return (h, c)

        z = jnp.zeros((SENT, H2), jnp.float32)
        lax.fori_loop(0, TLEN, step, (z, z))

    run(wif_ref, whf_ref, bf_ref, outf_ref, False)
    run(wib_ref, whb_ref, bb_ref, outb_ref, True)


def _run_lstm(embT, W_ih_f, W_hh_f, b_f, W_ih_b, W_hh_b, b_b):
    H2 = HID // 2
    out_shapes = (
        jax.ShapeDtypeStruct((TLEN, SENT, H2), jnp.float32),
        jax.ShapeDtypeStruct((TLEN, SENT, H2), jnp.float32),
    )
    return pl.pallas_call(
        _lstm_body,
        out_shape=out_shapes,
    )(embT.reshape(TLEN, SENT, EMB), W_ih_f.T, W_hh_f.T, b_f.reshape(1, -1),
      W_ih_b.T, W_hh_b.T, b_b.reshape(1, -1))


_AGG_BLK = 400
_AGG_NBLK = 2 * N_NODES // _AGG_BLK


def _agg0_body(h_ref, neigh_ref, w_ref, out_ref, len_ref):
    neigh = neigh_ref[...].astype(jnp.float32)
    r = jnp.sum(jax.nn.relu(neigh), axis=2)
    lens = jnp.sum(jnp.sign(r), axis=1, keepdims=True)
    len_ref[...] = lens
    s = jnp.sum(neigh, axis=1)
    means = s / jnp.maximum(lens, 1.0)
    w = w_ref[0]
    acc = (jnp.dot(h_ref[...], w[0:HID, :], preferred_element_type=jnp.float32)
           + jnp.dot(means, w[HID:2 * HID, :], preferred_element_type=jnp.float32))
    out_ref[...] = jax.nn.relu(acc)


def _aggk_body(h_ref, sums_ref, len_ref, w_ref, out_ref):
    means = sums_ref[...] / jnp.maximum(len_ref[...], 1.0)
    w = w_ref[0]
    acc = (jnp.dot(h_ref[...], w[0:HID, :], preferred_element_type=jnp.float32)
           + jnp.dot(means, w[HID:2 * HID, :], preferred_element_type=jnp.float32))
    out_ref[...] = jax.nn.relu(acc)


_HSPEC = pl.BlockSpec((_AGG_BLK, HID), lambda i: (i, 0))
_NSPEC = pl.BlockSpec((_AGG_BLK, ADJ_W, HID), lambda i: (i, 0, 0))
_LSPEC = pl.BlockSpec((_AGG_BLK, 1), lambda i: (i, 0))
_WSPEC = pl.BlockSpec((1, 2 * HID, HID), lambda i: (i // (_AGG_NBLK // 2), 0, 0))


def _agg_layer0(h_cat, neigh3, Wcat):
    return pl.pallas_call(
        _agg0_body,
        grid=(_AGG_NBLK,),
        in_specs=[_HSPEC, _NSPEC, _WSPEC],
        out_specs=[_HSPEC, _LSPEC],
        out_shape=[
            jax.ShapeDtypeStruct((2 * N_NODES, HID), jnp.float32),
            jax.ShapeDtypeStruct((2 * N_NODES, 1), jnp.float32),
        ],
    )(h_cat, neigh3, Wcat)


def _agg_layerk(h_cat, sums, lens, Wcat):
    return pl.pallas_call(
        _aggk_body,
        grid=(_AGG_NBLK,),
        in_specs=[_HSPEC, _HSPEC, _LSPEC, _WSPEC],
        out_specs=_HSPEC,
        out_shape=jax.ShapeDtypeStruct((2 * N_NODES, HID), jnp.float32),
    )(h_cat, sums, lens, Wcat)


def _pool_body(fw_ref, bw_ref, out_ref):
    pf = jnp.max(fw_ref[...], axis=1)
    pb = jnp.max(bw_ref[...], axis=1)
    out_ref[...] = jnp.concatenate([pf, pb], axis=-1)


def _run_pool(fw3, bw3):
    nb = fw3.shape[0]
    return pl.pallas_call(
        _pool_body,
        out_shape=jax.ShapeDtypeStruct((nb, 2 * HID), jnp.float32),
    )(fw3, bw3)


def _remap(n):
    return (n % SENT) * TLEN + n // SENT


def kernel(fw_adj_info, bw_adj_info, feature_info, batch_nodes, embed_table,
           W_ih_f, W_hh_f, b_f, W_ih_b, W_hh_b, b_b, fw_agg_W, bw_agg_W):
    nodes = batch_nodes.reshape(-1).astype(jnp.int32)
    Nb = nodes.shape[0]

    idxT = feature_info.T.reshape(-1).astype(jnp.int32)
    embT = _gather_rows(embed_table, idxT)

    out_f, out_b = _run_lstm(embT, W_ih_f, W_hh_f, b_f, W_ih_b, W_hh_b, b_b)
    table_pm = jnp.concatenate(
        [out_f.reshape(-1, HID // 2), out_b.reshape(-1, HID // 2)],
        axis=-1)
    output_vector = table_pm.reshape(TLEN, SENT, HID).swapaxes(0, 1)

    pad16 = ((Nb + 16 * NW - 1) // (16 * NW)) * (16 * NW)
    bias = jnp.int32(0x4B000000)
    adj_f = jnp.pad(fw_adj_info.astype(jnp.int32) + bias,
                    ((0, 0), (0, 128 - ADJ_W)))
    adj_b = jnp.pad(bw_adj_info.astype(jnp.int32) + bias,
                    ((0, 0), (0, 128 - ADJ_W)))
    big_table = jnp.concatenate(
        [table_pm,
         lax.bitcast_convert_type(adj_f, jnp.float32),
         lax.bitcast_convert_type(adj_b, jnp.float32)])
    def seg(i, off):
        i = jnp.concatenate([i, jnp.zeros((pad16 - Nb,), jnp.int32)])
        return i + jnp.int32(off)
    combo_idx = jnp.concatenate(
        [seg(_remap(nodes), 0), seg(nodes, N_NODES), seg(nodes, 2 * N_NODES)])
    combo = _gather_rows(big_table, combo_idx)
    h0 = combo[:Nb]
    fw_flat = lax.bitcast_convert_type(
        combo[pad16:pad16 + Nb, :ADJ_W], jnp.int32).reshape(-1) - bias
    bw_flat = lax.bitcast_convert_type(
        combo[2 * pad16:2 * pad16 + Nb, :ADJ_W], jnp.int32).reshape(-1) - bias

    idx0 = jnp.concatenate([_remap(fw_flat), _remap(bw_flat)])
    idxk = jnp.concatenate([fw_flat, bw_flat + N_NODES])
    h_cat = jnp.concatenate([h0, h0])
    lens = None
    for layer in range(LAYERS):
        Wcat = jnp.stack([fw_agg_W[layer], bw_agg_W[layer]])
        if layer == 0:
            neigh = _gather_rows(table_pm, idx0, big=True)
            neigh3 = neigh.reshape(2 * Nb, ADJ_W, HID)
            h_cat, lens = _agg_layer0(h_cat, neigh3, Wcat)
        else:
            sums = jnp.concatenate(
                [_gather_sum_rows(h_cat, fw_flat),
                 _gather_sum_rows(h_cat, bw_flat + N_NODES)])
            h_cat = _agg_layerk(h_cat, sums, lens, Wcat)

    nb_rows, nb_cols = batch_nodes.shape
    fw3 = h_cat[:Nb].reshape(nb_rows, nb_cols, HID)
    bw3 = h_cat[Nb:].reshape(nb_rows, nb_cols, HID)
    pooled = _run_pool(fw3, bw3)
    hidden = jnp.concatenate([fw3, bw3], axis=2)
    graph_embedding = pooled.reshape(-1, HID)
    return hidden, graph_embedding, output_vector


# speedup vs baseline: 1.0716x; 1.0716x over previous
import functools

import jax
import jax.numpy as jnp
from jax import lax
from jax.experimental import pallas as pl
from jax.experimental.pallas import tpu as pltpu
from jax.experimental.pallas import tpu_sc as plsc

N_NODES = 10000
ADJ_W = 32
EMB = 128
HID = 128
LAYERS = 3
SENT = 100
TLEN = 100

NC = 2
NS = 16
NW = NC * NS


def _sc_gather_small(V, D, n_chunks, chunk, dtype):
    rows_pw = n_chunks * chunk
    mesh = plsc.VectorSubcoreMesh(core_axis_name="c", subcore_axis_name="s")

    @functools.partial(
        pl.kernel,
        out_type=jax.ShapeDtypeStruct((NW * rows_pw, D), dtype),
        mesh=mesh,
        scratch_types=[
            pltpu.VMEM((n_chunks, chunk), jnp.int32),
            pltpu.VMEM((rows_pw, D), dtype),
            pltpu.SemaphoreType.DMA,
        ],
    )
    def k(table_h, idx_h, out_h, idx_v, rows_v, sem):
        wid = lax.axis_index("s") * NC + lax.axis_index("c")
        base = wid * rows_pw
        pltpu.sync_copy(idx_h.at[wid], idx_v)
        handles = []
        for ci in range(n_chunks):
            handles.append(
                pltpu.async_copy(
                    table_h.at[idx_v.at[ci]],
                    rows_v.at[pl.ds(ci * chunk, chunk)],
                    sem,
                )
            )
        for h in handles:
            h.wait()
        pltpu.sync_copy(rows_v, out_h.at[pl.ds(base, rows_pw)])

    return k


def _sc_gather_big(V, D, n_chunks, chunk, dtype):
    rows_pw = n_chunks * chunk
    mesh = plsc.VectorSubcoreMesh(core_axis_name="c", subcore_axis_name="s")
    NB = 2
    assert n_chunks % NB == 0
    groups = n_chunks // NB
    assert groups >= 2

    assert groups % 2 == 0

    @functools.partial(
        pl.kernel,
        out_type=jax.ShapeDtypeStruct((NW * rows_pw, D), dtype),
        mesh=mesh,
        scratch_types=[
            pltpu.VMEM((n_chunks, chunk), jnp.int32),
            pltpu.VMEM((2 * NB, chunk, D), dtype),
            pltpu.SemaphoreType.DMA,
            pltpu.SemaphoreType.DMA,
            pltpu.SemaphoreType.DMA,
        ],
    )
    def k(table_h, idx_h, out_h, idx_v, rows_v, gsem, osem_a, osem_b):
        wid = lax.axis_index("s") * NC + lax.axis_index("c")
        base = wid * rows_pw
        pltpu.sync_copy(idx_h.at[wid], idx_v)

        def gfire(ci, buf):
            pltpu.async_copy(table_h.at[idx_v.at[ci]], rows_v.at[buf], gsem)

        def gdrain(buf):
            pltpu.make_async_copy(
                table_h.at[idx_v.at[0]], rows_v.at[buf], gsem).wait()

        def ofire(ci, buf, osem):
            pltpu.async_copy(
                rows_v.at[buf], out_h.at[pl.ds(base + ci * chunk, chunk)], osem)

        def odrain(buf, osem):
            pltpu.make_async_copy(
                rows_v.at[buf], out_h.at[pl.ds(base, chunk)], osem).wait()

        for b in range(NB):
            gfire(b, b)

        def seg(g, s, osem_this, osem_other):
            c0 = g * NB
            o = NB - s
            for b in range(NB):
                gdrain(s + b)
            for b in range(NB):
                ofire(c0 + b, s + b, osem_this)
            @pl.when(g > 0)
            def _():
                for b in range(NB):
                    odrain(o + b, osem_other)
            @pl.when(g < groups - 1)
            def _():
                for b in range(NB):
                    gfire(c0 + NB + b, o + b)

        def body(g, carry):
            even = lax.rem(g, 2) == 0
            @pl.when(even)
            def _():
                seg(g, 0, osem_a, osem_b)
            @pl.when(jnp.logical_not(even))
            def _():
                seg(g, NB, osem_b, osem_a)
            return carry

        lax.fori_loop(0, groups, body, 0)
        for b in range(NB):
            odrain(NB + b, osem_b)

    return k


def _sc_gather_sum(V, n_chunks, chunk):
    D = HID
    rows_pw = n_chunks * chunk
    rows_out = rows_pw // ADJ_W
    mesh = plsc.VectorSubcoreMesh(core_axis_name="c", subcore_axis_name="s")
    NB = 2
    assert n_chunks % (2 * NB) == 0
    groups = n_chunks // NB

    @functools.partial(
        pl.kernel,
        out_type=jax.ShapeDtypeStruct((NW * rows_out, D), jnp.float32),
        mesh=mesh,
        scratch_types=[
            pltpu.VMEM((n_chunks, chunk), jnp.int32),
            pltpu.VMEM((n_chunks, chunk), jnp.int32),
            pltpu.VMEM((2 * NB, chunk, D), jnp.float32),
            pltpu.VMEM_SHARED((NS * rows_out, D), jnp.float32),
            pltpu.SemaphoreType.DMA,
            pltpu.SemaphoreType.DMA,
            pltpu.SemaphoreType.DMA,
        ],
    )
    def k(table_h, idx_h, scat_h, zer_h, out_h, idx_v, scat_v, rows_v, acc_s,
          gsem, osem_a, osem_b):
        cid = lax.axis_index("c")
        sid = lax.axis_index("s")
        wid = sid * NC + cid
        pltpu.sync_copy(idx_h.at[wid], idx_v)
        pltpu.sync_copy(scat_h.at[wid], scat_v)
        pltpu.sync_copy(zer_h, acc_s.at[pl.ds(sid * rows_out, rows_out)])

        def gfire(ci, buf):
            pltpu.async_copy(table_h.at[idx_v.at[ci]], rows_v.at[buf], gsem)

        def gdrain(buf):
            pltpu.make_async_copy(
                table_h.at[idx_v.at[0]], rows_v.at[buf], gsem).wait()

        def ofire(ci, buf, osem):
            pltpu.async_copy(
                rows_v.at[buf], acc_s.at[scat_v.at[ci]], osem, add=True)

        def odrain(buf, osem):
            pltpu.make_async_copy(
                rows_v.at[buf], acc_s.at[scat_v.at[0]], osem).wait()

        for b in range(NB):
            gfire(b, b)

        def seg(g, s, osem_this, osem_other):
            c0 = g * NB
            o = NB - s
            for b in range(NB):
                gdrain(s + b)
            for b in range(NB):
                ofire(c0 + b, s + b, osem_this)
            @pl.when(g > 0)
            def _():
                for b in range(NB):
                    odrain(o + b, osem_other)
            @pl.when(g < groups - 1)
            def _():
                for b in range(NB):
                    gfire(c0 + NB + b, o + b)

        def body(g, carry):
            even = lax.rem(g, 2) == 0
            @pl.when(even)
            def _():
                seg(g, 0, osem_a, osem_b)
            @pl.when(jnp.logical_not(even))
            def _():
                seg(g, NB, osem_b, osem_a)
            return carry

        lax.fori_loop(0, groups, body, 0)
        for b in range(NB):
            odrain(NB + b, osem_b)
        pltpu.sync_copy(acc_s.at[pl.ds(sid * rows_out, rows_out)],
                        out_h.at[pl.ds(wid * rows_out, rows_out)])

    return k


def _gather_sum_rows(table, idx_flat):
    B = idx_flat.shape[0]
    V, D = table.shape
    chunk = 128
    n_chunks = -(-B // (NW * chunk))
    n_chunks = ((n_chunks + 3) // 4) * 4
    rows_pw = n_chunks * chunk
    rows_out = rows_pw // ADJ_W
    fn = _sc_gather_sum(V, n_chunks, chunk)
    idx3 = _pad_idx(idx_flat, n_chunks, chunk)
    base = (jnp.arange(rows_pw, dtype=jnp.int32) // ADJ_W).reshape(
        n_chunks, chunk)
    scat3 = base[None] + ((jnp.arange(NW, dtype=jnp.int32) // NC)
                          * rows_out)[:, None, None]
    zer = jnp.zeros((rows_out, D), jnp.float32)
    out = fn(table, idx3, scat3, zer)
    return out[:B // ADJ_W]


def _pad_idx(idx_flat, n_chunks, chunk):
    total = NW * n_chunks * chunk
    idx_flat = idx_flat.astype(jnp.int32)
    pad = total - idx_flat.shape[0]
    if pad:
        idx_flat = jnp.concatenate([idx_flat, jnp.zeros((pad,), jnp.int32)])
    return idx_flat.reshape(NW, n_chunks, chunk)


def _gather_rows(table, idx_flat, big=False):
    B = idx_flat.shape[0]
    V, D = table.shape
    chunk = 128 if big else 64
    n_chunks = -(-B // (NW * chunk))
    if big:
        n_chunks = ((n_chunks + 3) // 4) * 4
        fn = _sc_gather_big(V, D, n_chunks, chunk, table.dtype)
    else:
        fn = _sc_gather_small(V, D, n_chunks, chunk, table.dtype)
    idx3 = _pad_idx(idx_flat, n_chunks, chunk)
    out = fn(table, idx3)
    return out[:B]


def _lstm_body(emb_ref, wif_ref, whf_ref, bf_ref, wib_ref, whb_ref, bb_ref,
               outf_ref, outb_ref):
    H2 = HID // 2

    def run(wi_ref, wh_ref, b_ref, out_ref, reverse):
        def step(s, carry):
            h, c = carry
            t = (TLEN - 1 - s) if reverse else s
            xt = emb_ref[t]
            g = (jnp.dot(xt, wi_ref[...], preferred_element_type=jnp.float32)
                 + jnp.dot(h, wh_ref[...], preferred_element_type=jnp.float32)
                 + b_ref[...])
            i = jax.nn.sigmoid(g[:, 0:H2])
            f = jax.nn.sigmoid(g[:, H2:2 * H2])
            gg = jnp.tanh(g[:, 2 * H2:3 * H2])
            o = jax.nn.sigmoid(g[:, 3 * H2:4 * H2])
            c = f * c + i * gg
            h = o * jnp.tanh(c)
            out_ref[t] = h
            return (h, c)

        z = jnp.zeros((SENT, H2), jnp.float32)
        lax.fori_loop(0, TLEN, step, (z, z))

    run(wif_ref, whf_ref, bf_ref, outf_ref, False)
    run(wib_ref, whb_ref, bb_ref, outb_ref, True)


def _run_lstm(embT, W_ih_f, W_hh_f, b_f, W_ih_b, W_hh_b, b_b):
    H2 = HID // 2
    out_shapes = (
        jax.ShapeDtypeStruct((TLEN, SENT, H2), jnp.float32),
        jax.ShapeDtypeStruct((TLEN, SENT, H2), jnp.float32),
    )
    return pl.pallas_call(
        _lstm_body,
        out_shape=out_shapes,
    )(embT.reshape(TLEN, SENT, EMB), W_ih_f.T, W_hh_f.T, b_f.reshape(1, -1),
      W_ih_b.T, W_hh_b.T, b_b.reshape(1, -1))


_AGG_BLK = 400
_AGG_NBLK = 2 * N_NODES // _AGG_BLK


def _agg0_body(h_ref, neigh_ref, w_ref, out_ref, len_ref):
    neigh = neigh_ref[...]
    r = jnp.sum(jax.nn.relu(neigh), axis=2)
    lens = jnp.sum(jnp.sign(r), axis=1, keepdims=True)
    len_ref[...] = lens
    s = jnp.sum(neigh, axis=1)
    means = s / jnp.maximum(lens, 1.0)
    w = w_ref[0]
    acc = (jnp.dot(h_ref[...], w[0:HID, :], preferred_element_type=jnp.float32)
           + jnp.dot(means, w[HID:2 * HID, :], preferred_element_type=jnp.float32))
    out_ref[...] = jax.nn.relu(acc)


def _aggk_body(h_ref, sums_ref, len_ref, w_ref, out_ref):
    means = sums_ref[...] / jnp.maximum(len_ref[...], 1.0)
    w = w_ref[0]
    acc = (jnp.dot(h_ref[...], w[0:HID, :], preferred_element_type=jnp.float32)
           + jnp.dot(means, w[HID:2 * HID, :], preferred_element_type=jnp.float32))
    out_ref[...] = jax.nn.relu(acc)


_HSPEC = pl.BlockSpec((_AGG_BLK, HID), lambda i: (i, 0))
_NSPEC = pl.BlockSpec((_AGG_BLK, ADJ_W, HID), lambda i: (i, 0, 0))
_LSPEC = pl.BlockSpec((_AGG_BLK, 1), lambda i: (i, 0))
_WSPEC = pl.BlockSpec((1, 2 * HID, HID), lambda i: (i // (_AGG_NBLK // 2), 0, 0))


def _agg_layer0(h_cat, neigh3, Wcat):
    return pl.pallas_call(
        _agg0_body,
        grid=(_AGG_NBLK,),
        in_specs=[_HSPEC, _NSPEC, _WSPEC],
        out_specs=[_HSPEC, _LSPEC],
        out_shape=[
            jax.ShapeDtypeStruct((2 * N_NODES, HID), jnp.float32),
            jax.ShapeDtypeStruct((2 * N_NODES, 1), jnp.float32),
        ],
    )(h_cat, neigh3, Wcat)


def _agg_layerk(h_cat, sums, lens, Wcat):
    return pl.pallas_call(
        _aggk_body,
        grid=(_AGG_NBLK,),
        in_specs=[_HSPEC, _HSPEC, _LSPEC, _WSPEC],
        out_specs=_HSPEC,
        out_shape=jax.ShapeDtypeStruct((2 * N_NODES, HID), jnp.float32),
    )(h_cat, sums, lens, Wcat)


def _pool_body(fw_ref, bw_ref, out_ref):
    pf = jnp.max(fw_ref[...], axis=1)
    pb = jnp.max(bw_ref[...], axis=1)
    out_ref[...] = jnp.concatenate([pf, pb], axis=-1)


def _run_pool(fw3, bw3):
    nb = fw3.shape[0]
    return pl.pallas_call(
        _pool_body,
        out_shape=jax.ShapeDtypeStruct((nb, 2 * HID), jnp.float32),
    )(fw3, bw3)


def _remap(n):
    return (n % SENT) * TLEN + n // SENT


def kernel(fw_adj_info, bw_adj_info, feature_info, batch_nodes, embed_table,
           W_ih_f, W_hh_f, b_f, W_ih_b, W_hh_b, b_b, fw_agg_W, bw_agg_W):
    nodes = batch_nodes.reshape(-1).astype(jnp.int32)
    Nb = nodes.shape[0]

    idxT = feature_info.T.reshape(-1).astype(jnp.int32)
    embT = _gather_rows(embed_table, idxT)

    out_f, out_b = _run_lstm(embT, W_ih_f, W_hh_f, b_f, W_ih_b, W_hh_b, b_b)
    table_pm = jnp.concatenate(
        [out_f.reshape(-1, HID // 2), out_b.reshape(-1, HID // 2)],
        axis=-1)
    output_vector = table_pm.reshape(TLEN, SENT, HID).swapaxes(0, 1)

    pad16 = ((Nb + 16 * NW - 1) // (16 * NW)) * (16 * NW)
    bias = jnp.int32(0x4B000000)
    adj_f = jnp.pad(fw_adj_info.astype(jnp.int32) + bias,
                    ((0, 0), (0, 128 - ADJ_W)))
    adj_b = jnp.pad(bw_adj_info.astype(jnp.int32) + bias,
                    ((0, 0), (0, 128 - ADJ_W)))
    big_table = jnp.concatenate(
        [table_pm,
         lax.bitcast_convert_type(adj_f, jnp.float32),
         lax.bitcast_convert_type(adj_b, jnp.float32)])
    def seg(i, off):
        i = jnp.concatenate([i, jnp.zeros((pad16 - Nb,), jnp.int32)])
        return i + jnp.int32(off)
    combo_idx = jnp.concatenate(
        [seg(_remap(nodes), 0), seg(nodes, N_NODES), seg(nodes, 2 * N_NODES)])
    combo = _gather_rows(big_table, combo_idx)
    h0 = combo[:Nb]
    fw_flat = lax.bitcast_convert_type(
        combo[pad16:pad16 + Nb, :ADJ_W], jnp.int32).reshape(-1) - bias
    bw_flat = lax.bitcast_convert_type(
        combo[2 * pad16:2 * pad16 + Nb, :ADJ_W], jnp.int32).reshape(-1) - bias

    idx0 = jnp.concatenate([_remap(fw_flat), _remap(bw_flat)])
    idxk = jnp.concatenate([fw_flat, bw_flat + N_NODES])
    h_cat = jnp.concatenate([h0, h0])
    lens = None
    for layer in range(LAYERS):
        Wcat = jnp.stack([fw_agg_W[layer], bw_agg_W[layer]])
        if layer == 0:
            neigh = _gather_rows(table_pm, idx0, big=True)
            neigh3 = neigh.reshape(2 * Nb, ADJ_W, HID)
            h_cat, lens = _agg_layer0(h_cat, neigh3, Wcat)
        else:
            sums = jnp.concatenate(
                [_gather_sum_rows(h_cat, fw_flat),
                 _gather_sum_rows(h_cat, bw_flat + N_NODES)])
            h_cat = _agg_layerk(h_cat, sums, lens, Wcat)

    nb_rows, nb_cols = batch_nodes.shape
    fw3 = h_cat[:Nb].reshape(nb_rows, nb_cols, HID)
    bw3 = h_cat[Nb:].reshape(nb_rows, nb_cols, HID)
    pooled = _run_pool(fw3, bw3)
    hidden = jnp.concatenate([fw3, bw3], axis=2)
    graph_embedding = pooled.reshape(-1, HID)
    return hidden, graph_embedding, output_vector
